# baseline (speedup 1.0000x reference)
import jax
import jax.numpy as jnp
from jax import lax
from jax.experimental import pallas as pl
from jax.experimental.pallas import tpu as pltpu
from jax.experimental.pallas import tpu_sc as plsc

N = 10000
E = 160000
D = 256
P = 12
F = 64
NPAD = 10240
EP = 163840
K = 128
ECH = EP // K
NC = 2
NS = 16
RT = NPAD // NS
ER4 = ECH // NS
ER1 = ECH // (NC * NS)

_MESH = dict(core_axis_name="c", subcore_axis_name="s")
_SC_PARAMS = pltpu.CompilerParams(use_tc_tiling_on_sc=False,
                                  needs_layout_passes=False)


def _zero_2d(buf, rows, vecs):
    def body(r, _):
        for k in range(vecs):
            buf[r, pl.ds(k * 16, 16)] = jnp.zeros((16,), jnp.float32)
        return 0
    lax.fori_loop(0, rows, body, 0)


def _norm_inplace(dis_t, src_t, dst_t, norm_t, nrows):
    g = K // 16

    def body(t, _):
        j = t // g
        o = (t % g) * 16
        s16 = plsc.load_gather(dis_t, [src_t[j, pl.ds(o, 16)]])
        d16 = plsc.load_gather(dis_t, [dst_t[j, pl.ds(o, 16)]])
        ew16 = norm_t[j, pl.ds(o, 16)]
        norm_t[j, pl.ds(o, 16)] = -(s16 * d16 * ew16)
        return 0
    lax.fori_loop(0, nrows * g, body, 0)


def _scale_rows(gbuf, norm_t, ch, vecs):
    def body(g, _):
        nvec = norm_t[ch, pl.ds(g * 16, 16)]
        for l in range(16):
            nv = jnp.broadcast_to(nvec[l], (16,))
            e = g * 16 + l
            for k in range(vecs):
                gbuf[e, pl.ds(k * 16, 16)] = gbuf[e, pl.ds(k * 16, 16)] * nv
        return 0
    lax.fori_loop(0, K // 16, body, 0, unroll=4)


def _zero_panel(gb, sh, base):
    for i in range(RT // K):
        pltpu.sync_copy(gb, sh.at[pl.ds(base + i * K, K)])


def _spmm_pipeline(table_h, idx_t, dst_t, norm_t, gbs, gsems, ssems,
                   acc_sh, nchunks):
    for b in range(3):
        pltpu.async_copy(table_h.at[idx_t.at[b]], gbs[b], gsems[b])

    def body(i, _):
        base = 4 * i
        for j in range(4):
            ch = base + j
            pltpu.make_async_copy(table_h.at[idx_t.at[ch]], gbs[j],
                                  gsems[j]).wait()
            _scale_rows(gbs[j], norm_t, ch, F // 16)
            pltpu.async_copy(gbs[j], acc_sh.at[dst_t.at[ch]], ssems[j],
                             add=True)
            jj = (j + 3) % 4
            nxt = ch + 3
            prv = ch - 1
            if j == 0:
                @pl.when(i > 0)
                def _():
                    pltpu.make_async_copy(gbs[jj],
                                          acc_sh.at[dst_t.at[prv]],
                                          ssems[jj]).wait()
                pltpu.async_copy(table_h.at[idx_t.at[nxt]], gbs[jj],
                                 gsems[jj])
            else:
                @pl.when(nxt < nchunks)
                def _():
                    pltpu.make_async_copy(gbs[jj],
                                          acc_sh.at[dst_t.at[prv]],
                                          ssems[jj]).wait()
                    pltpu.async_copy(table_h.at[idx_t.at[nxt]], gbs[jj],
                                     gsems[jj])
        return 0
    lax.fori_loop(0, nchunks // 4, body, 0)
    for j in range(4):
        pltpu.make_async_copy(gbs[j], acc_sh.at[dst_t.at[nchunks - 4 + j]],
                              ssems[j]).wait()


def _spmm_body(u12_h, src_h, dst_h, ew_h, out_h, dis_o,
               dis_t, idx_t, dst_t, norm_t, zb,
               gb0, gb1, gb2, gb3, gs0, gs1, gs2, gs3,
               ss0, ss1, ss2, ss3, z_sh, deg_sh):
    c = lax.axis_index("c")
    w = lax.axis_index("s")
    row0 = w * ER4
    pltpu.sync_copy(src_h.at[pl.ds(row0, ER4)], idx_t)
    pltpu.sync_copy(dst_h.at[pl.ds(row0, ER4)], dst_t)
    pltpu.sync_copy(ew_h.at[pl.ds(row0, ER4)], norm_t)

    def zdeg(r, _):
        zb[pl.ds(r * 16, 16)] = jnp.zeros((16,), jnp.float32)
        return 0
    lax.fori_loop(0, RT // 16, zdeg, 0)
    pltpu.sync_copy(zb, deg_sh.at[pl.ds(w * RT, RT)])
    plsc.subcore_barrier()

    def degadd(j, _):
        pltpu.sync_copy(norm_t.at[j], deg_sh.at[idx_t.at[j]], add=True)
        return 0
    lax.fori_loop(0, ER4, degadd, 0)
    plsc.subcore_barrier()

    pltpu.sync_copy(deg_sh, dis_t)

    def disbody(t, _):
        d0 = dis_t[pl.ds(t * 16, 16)]
        pos = d0 > 0.0
        d = jnp.where(pos, d0, 1.0)
        iv = plsc.bitcast(d, jnp.int32)
        iv = 0x5F3759DF - lax.shift_right_logical(iv, 1)
        y = plsc.bitcast(iv, jnp.float32)
        hd = 0.5 * d
        for _n in range(3):
            y = y * (1.5 - hd * y * y)
        dis_t[pl.ds(t * 16, 16)] = jnp.where(pos, y, 0.0)
        return 0
    lax.fori_loop(0, NPAD // 16, disbody, 0)

    @pl.when(c == 0)
    def _():
        pltpu.sync_copy(dis_t.at[pl.ds(w * RT, RT)], dis_o.at[pl.ds(w * RT, RT)])

    _norm_inplace(dis_t, idx_t, dst_t, norm_t, ER4)

    def to_idx(t, _):
        j = t // (K // 16)
        o = (t % (K // 16)) * 16
        idx_t[j, pl.ds(o, 16)] = idx_t[j, pl.ds(o, 16)] * P + c * (P // NC)
        return 0
    lax.fori_loop(0, ER4 * (K // 16), to_idx, 0)

    _zero_2d(gb0, K, F // 16)
    _zero_panel(gb0, z_sh, w * RT)
    plsc.subcore_barrier()

    def panel_body(jp, _):
        p = c * (P // NC) + jp
        _spmm_pipeline(u12_h, idx_t, dst_t, norm_t,
                       (gb0, gb1, gb2, gb3), (gs0, gs1, gs2, gs3),
                       (ss0, ss1, ss2, ss3), z_sh, ER4)
        plsc.subcore_barrier()
        pltpu.sync_copy(z_sh.at[pl.ds(w * RT, RT)],
                        out_h.at[p, pl.ds(w * RT, RT)])

        def advance(t, __):
            j = t // (K // 16)
            o = (t % (K // 16)) * 16
            idx_t[j, pl.ds(o, 16)] = idx_t[j, pl.ds(o, 16)] + 1
            return 0
        lax.fori_loop(0, ER4 * (K // 16), advance, 0)
        _zero_2d(gb0, K, F // 16)
        _zero_panel(gb0, z_sh, w * RT)
        plsc.subcore_barrier()
        return 0

    lax.fori_loop(0, P // NC, panel_body, 0)


def _sc_spmm_panels(u12, src2, dst2, ew2):
    kfn = pl.kernel(
        _spmm_body,
        out_type=[jax.ShapeDtypeStruct((P, NPAD, F), jnp.float32),
                  jax.ShapeDtypeStruct((NPAD,), jnp.float32)],
        mesh=plsc.VectorSubcoreMesh(**_MESH),
        compiler_params=_SC_PARAMS,
        scratch_types=[
            pltpu.VMEM((NPAD,), jnp.float32),
            pltpu.VMEM((ER4, K), jnp.int32),
            pltpu.VMEM((ER4, K), jnp.int32),
            pltpu.VMEM((ER4, K), jnp.float32),
            pltpu.VMEM((RT,), jnp.float32),
            pltpu.VMEM((K, F), jnp.float32),
            pltpu.VMEM((K, F), jnp.float32),
            pltpu.VMEM((K, F), jnp.float32),
            pltpu.VMEM((K, F), jnp.float32),
            pltpu.SemaphoreType.DMA,
            pltpu.SemaphoreType.DMA,
            pltpu.SemaphoreType.DMA,
            pltpu.SemaphoreType.DMA,
            pltpu.SemaphoreType.DMA,
            pltpu.SemaphoreType.DMA,
            pltpu.SemaphoreType.DMA,
            pltpu.SemaphoreType.DMA,
            pltpu.VMEM_SHARED((NPAD, F), jnp.float32),
            pltpu.VMEM_SHARED((NPAD,), jnp.float32),
        ],
    )
    return kfn(u12, src2, dst2, ew2)


def _spmm1_body(s_h, src_h, dst_h, ew_h, dis_h, out_h,
                dis_t, src_t, dst_t, norm_t,
                gb0, gb1, gb2, gb3, gs0, gs1, gs2, gs3,
                ss0, ss1, ss2, ss3, t_sh):
    c = lax.axis_index("c")
    w = lax.axis_index("s")
    row0 = (c * NS + w) * ER1
    pltpu.sync_copy(dis_h, dis_t)
    pltpu.sync_copy(src_h.at[pl.ds(row0, ER1)], src_t)
    pltpu.sync_copy(dst_h.at[pl.ds(row0, ER1)], dst_t)
    pltpu.sync_copy(ew_h.at[pl.ds(row0, ER1)], norm_t)
    _norm_inplace(dis_t, src_t, dst_t, norm_t, ER1)

    _zero_2d(gb0, K, F // 16)
    _zero_panel(gb0, t_sh, w * RT)
    plsc.subcore_barrier()
    _spmm_pipeline(s_h, src_t, dst_t, norm_t,
                   (gb0, gb1, gb2, gb3), (gs0, gs1, gs2, gs3),
                   (ss0, ss1, ss2, ss3), t_sh, ER1)
    plsc.subcore_barrier()
    pltpu.sync_copy(t_sh.at[pl.ds(w * RT, RT)],
                    out_h.at[c, pl.ds(w * RT, RT)])


def _sc_spmm_single(s, src2, dst2, ew2, dis):
    kfn = pl.kernel(
        _spmm1_body,
        out_type=jax.ShapeDtypeStruct((NC, NPAD, F), jnp.float32),
        mesh=plsc.VectorSubcoreMesh(**_MESH),
        compiler_params=_SC_PARAMS,
        scratch_types=[
            pltpu.VMEM((NPAD,), jnp.float32),
            pltpu.VMEM((ER1, K), jnp.int32),
            pltpu.VMEM((ER1, K), jnp.int32),
            pltpu.VMEM((ER1, K), jnp.float32),
            pltpu.VMEM((K, F), jnp.float32),
            pltpu.VMEM((K, F), jnp.float32),
            pltpu.VMEM((K, F), jnp.float32),
            pltpu.VMEM((K, F), jnp.float32),
            pltpu.SemaphoreType.DMA,
            pltpu.SemaphoreType.DMA,
            pltpu.SemaphoreType.DMA,
            pltpu.SemaphoreType.DMA,
            pltpu.SemaphoreType.DMA,
            pltpu.SemaphoreType.DMA,
            pltpu.SemaphoreType.DMA,
            pltpu.SemaphoreType.DMA,
            pltpu.VMEM_SHARED((NPAD, F), jnp.float32),
        ],
    )
    return kfn(s, src2, dst2, ew2, dis)


NB = 1000


def _k1_body(xt_ref, w_ref, u_ref):
    wm = w_ref[...]
    for p in range(P):
        y = lax.dot_general(xt_ref[p], wm, (((1,), (1,)), ((), ())),
                            preferred_element_type=jnp.float32)
        u_ref[:, p, :] = y


def _tc_proj(xt, w):
    return pl.pallas_call(
        _k1_body,
        grid=(N // NB,),
        in_specs=[
            pl.BlockSpec((P, NB, D), lambda i: (0, i, 0)),
            pl.BlockSpec((F, D), lambda i: (0, 0)),
        ],
        out_specs=pl.BlockSpec((NB, P, F), lambda i: (i, 0, 0)),
        out_shape=jax.ShapeDtypeStruct((N, P, F), jnp.float32),
    )(xt, w)


def _k5_body(v_ref, z_ref, b1_ref, s_ref):
    b1 = b1_ref[...]
    acc = jnp.zeros((NB, F), jnp.float32)
    for p in range(P):
        acc = acc + jax.nn.relu(v_ref[:, p, :] + z_ref[p] + b1)
    s_ref[...] = acc


def _tc_sum(v3, z12, b1):
    return pl.pallas_call(
        _k5_body,
        grid=(N // NB,),
        in_specs=[
            pl.BlockSpec((NB, P, F), lambda i: (i, 0, 0)),
            pl.BlockSpec((P, NB, F), lambda i: (0, i, 0)),
            pl.BlockSpec((1, F), lambda i: (0, 0)),
        ],
        out_specs=pl.BlockSpec((NB, F), lambda i: (i, 0)),
        out_shape=jax.ShapeDtypeStruct((N, F), jnp.float32),
    )(v3, z12, b1)


def _k7_body(s_ref, t_ref, w20_ref, w21_ref, b2_ref, l1w_ref, l1b_ref,
             l2w_ref, l2b_ref, h_ref, y_ref):
    s = s_ref[...]
    t = t_ref[0] + t_ref[1]
    hh = (lax.dot_general(s, w20_ref[...], (((1,), (1,)), ((), ())),
                          preferred_element_type=jnp.float32)
          + lax.dot_general(t, w21_ref[...], (((1,), (1,)), ((), ())),
                            preferred_element_type=jnp.float32)
          + P * b2_ref[...])
    a1 = jax.nn.relu(
        lax.dot_general(hh, l1w_ref[...], (((1,), (1,)), ((), ())),
                        preferred_element_type=jnp.float32) + l1b_ref[...])
    y = lax.dot_general(a1, l2w_ref[...], (((1,), (1,)), ((), ())),
                        preferred_element_type=jnp.float32) + l2b_ref[...]
    h_ref[...] = hh
    y_ref[...] = y


def _tc_head(s, t2, w20, w21, b2, l1w, l1b, l2w, l2b):
    def full(shape):
        return pl.BlockSpec(shape, lambda i, _s=shape: tuple(0 for _ in _s))
    return pl.pallas_call(
        _k7_body,
        grid=(N // NB,),
        in_specs=[
            pl.BlockSpec((NB, F), lambda i: (i, 0)),
            pl.BlockSpec((NC, NB, F), lambda i: (0, i, 0)),
            full((D, F)), full((D, F)), full((1, D)),
            full((128, D)), full((1, 128)),
            full((P, 128)), full((1, P)),
        ],
        out_specs=[
            pl.BlockSpec((NB, D), lambda i: (i, 0)),
            pl.BlockSpec((NB, P), lambda i: (i, 0)),
        ],
        out_shape=[
            jax.ShapeDtypeStruct((N, D), jnp.float32),
            jax.ShapeDtypeStruct((N, P), jnp.float32),
        ],
    )(s, t2, w20, w21, b2, l1w, l1b, l2w, l2b)


@jax.jit
def kernel(x, edge_index, edge_attr, W10, W11, b1, W20, W21, b2, L1w, L1b, L2w, L2b):
    src = edge_index[0].astype(jnp.int32)
    dst = edge_index[1].astype(jnp.int32)
    pad = EP - E
    pidx = (jnp.arange(pad, dtype=jnp.int32) % N)
    src2 = jnp.concatenate([src, pidx]).reshape(ECH, K)
    dst2 = jnp.concatenate([dst, pidx]).reshape(ECH, K)
    ew2 = jnp.concatenate([edge_attr, jnp.zeros((pad,), jnp.float32)]).reshape(ECH, K)

    xt = jnp.transpose(x, (2, 0, 1))
    u3 = _tc_proj(xt, W11)
    u12 = u3.reshape(P * N, F)

    z12, dis = _sc_spmm_panels(u12, src2, dst2, ew2)
    v3 = _tc_proj(xt, W10)
    s = _tc_sum(v3, z12, b1.reshape(1, F))
    t2 = _sc_spmm_single(s, src2, dst2, ew2, dis)
    h, y = _tc_head(s, t2, W20, W21, b2.reshape(1, D),
                    L1w, L1b.reshape(1, 128), L2w, L2b.reshape(1, P))
    return (y, h)


# speedup vs baseline: 1.0025x; 1.0025x over previous
import jax
import jax.numpy as jnp
from jax import lax
from jax.experimental import pallas as pl
from jax.experimental.pallas import tpu as pltpu
from jax.experimental.pallas import tpu_sc as plsc

N = 10000
E = 160000
D = 256
P = 12
F = 64
NPAD = 10240
EP = 163840
K = 128
ECH = EP // K
NC = 2
NS = 16
RT = NPAD // NS
ER4 = ECH // NS
ER1 = ECH // (NC * NS)

_MESH = dict(core_axis_name="c", subcore_axis_name="s")
_SC_PARAMS = pltpu.CompilerParams(use_tc_tiling_on_sc=False,
                                  needs_layout_passes=False)


def _zero_2d(buf, rows, vecs):
    def body(r, _):
        for k in range(vecs):
            buf[r, pl.ds(k * 16, 16)] = jnp.zeros((16,), jnp.float32)
        return 0
    lax.fori_loop(0, rows, body, 0)


def _norm_inplace(dis_t, src_t, dst_t, norm_t, nrows):
    g = K // 16

    def body(t, _):
        j = t // g
        o = (t % g) * 16
        s16 = plsc.load_gather(dis_t, [src_t[j, pl.ds(o, 16)]])
        d16 = plsc.load_gather(dis_t, [dst_t[j, pl.ds(o, 16)]])
        ew16 = norm_t[j, pl.ds(o, 16)]
        norm_t[j, pl.ds(o, 16)] = -(s16 * d16 * ew16)
        return 0
    lax.fori_loop(0, nrows * g, body, 0)


def _scale_rows(gbuf, norm_t, ch, vecs):
    def body(g, _):
        nvec = norm_t[ch, pl.ds(g * 16, 16)]
        for l in range(16):
            nv = jnp.broadcast_to(nvec[l], (16,))
            e = g * 16 + l
            for k in range(vecs):
                gbuf[e, pl.ds(k * 16, 16)] = gbuf[e, pl.ds(k * 16, 16)] * nv
        return 0
    lax.fori_loop(0, K // 16, body, 0, unroll=2)


def _zero_panel(gb, sh, base):
    for i in range(RT // K):
        pltpu.sync_copy(gb, sh.at[pl.ds(base + i * K, K)])


def _spmm_pipeline(table_h, idx_t, dst_t, norm_t, gbs, gsems, ssems,
                   acc_sh, nchunks):
    for b in range(3):
        pltpu.async_copy(table_h.at[idx_t.at[b]], gbs[b], gsems[b])

    def body(i, _):
        base = 4 * i
        for j in range(4):
            ch = base + j
            pltpu.make_async_copy(table_h.at[idx_t.at[ch]], gbs[j],
                                  gsems[j]).wait()
            _scale_rows(gbs[j], norm_t, ch, F // 16)
            pltpu.async_copy(gbs[j], acc_sh.at[dst_t.at[ch]], ssems[j],
                             add=True)
            jj = (j + 3) % 4
            nxt = ch + 3
            prv = ch - 1
            if j == 0:
                @pl.when(i > 0)
                def _():
                    pltpu.make_async_copy(gbs[jj],
                                          acc_sh.at[dst_t.at[prv]],
                                          ssems[jj]).wait()
                pltpu.async_copy(table_h.at[idx_t.at[nxt]], gbs[jj],
                                 gsems[jj])
            else:
                @pl.when(nxt < nchunks)
                def _():
                    pltpu.make_async_copy(gbs[jj],
                                          acc_sh.at[dst_t.at[prv]],
                                          ssems[jj]).wait()
                    pltpu.async_copy(table_h.at[idx_t.at[nxt]], gbs[jj],
                                     gsems[jj])
        return 0
    lax.fori_loop(0, nchunks // 4, body, 0)
    for j in range(4):
        pltpu.make_async_copy(gbs[j], acc_sh.at[dst_t.at[nchunks - 4 + j]],
                              ssems[j]).wait()


def _spmm_body(u12_h, src_h, dst_h, ew_h, out_h, dis_o,
               dis_t, idx_t, dst_t, norm_t, zb,
               gb0, gb1, gb2, gb3, gs0, gs1, gs2, gs3,
               ss0, ss1, ss2, ss3, z_sh, deg_sh):
    c = lax.axis_index("c")
    w = lax.axis_index("s")
    row0 = w * ER4
    pltpu.sync_copy(src_h.at[pl.ds(row0, ER4)], idx_t)
    pltpu.sync_copy(dst_h.at[pl.ds(row0, ER4)], dst_t)
    pltpu.sync_copy(ew_h.at[pl.ds(row0, ER4)], norm_t)

    def zdeg(r, _):
        zb[pl.ds(r * 16, 16)] = jnp.zeros((16,), jnp.float32)
        return 0
    lax.fori_loop(0, RT // 16, zdeg, 0)
    pltpu.sync_copy(zb, deg_sh.at[pl.ds(w * RT, RT)])
    plsc.subcore_barrier()

    def degadd(j, _):
        pltpu.sync_copy(norm_t.at[j], deg_sh.at[idx_t.at[j]], add=True)
        return 0
    lax.fori_loop(0, ER4, degadd, 0)
    plsc.subcore_barrier()

    pltpu.sync_copy(deg_sh, dis_t)

    def disbody(t, _):
        d0 = dis_t[pl.ds(t * 16, 16)]
        pos = d0 > 0.0
        d = jnp.where(pos, d0, 1.0)
        iv = plsc.bitcast(d, jnp.int32)
        iv = 0x5F3759DF - lax.shift_right_logical(iv, 1)
        y = plsc.bitcast(iv, jnp.float32)
        hd = 0.5 * d
        for _n in range(3):
            y = y * (1.5 - hd * y * y)
        dis_t[pl.ds(t * 16, 16)] = jnp.where(pos, y, 0.0)
        return 0
    lax.fori_loop(0, NPAD // 16, disbody, 0)

    @pl.when(c == 0)
    def _():
        pltpu.sync_copy(dis_t.at[pl.ds(w * RT, RT)], dis_o.at[pl.ds(w * RT, RT)])

    _norm_inplace(dis_t, idx_t, dst_t, norm_t, ER4)

    def to_idx(t, _):
        j = t // (K // 16)
        o = (t % (K // 16)) * 16
        idx_t[j, pl.ds(o, 16)] = idx_t[j, pl.ds(o, 16)] * P + c * (P // NC)
        return 0
    lax.fori_loop(0, ER4 * (K // 16), to_idx, 0)

    _zero_2d(gb0, K, F // 16)
    _zero_panel(gb0, z_sh, w * RT)
    plsc.subcore_barrier()

    def panel_body(jp, _):
        p = c * (P // NC) + jp
        _spmm_pipeline(u12_h, idx_t, dst_t, norm_t,
                       (gb0, gb1, gb2, gb3), (gs0, gs1, gs2, gs3),
                       (ss0, ss1, ss2, ss3), z_sh, ER4)
        plsc.subcore_barrier()
        pltpu.sync_copy(z_sh.at[pl.ds(w * RT, RT)],
                        out_h.at[p, pl.ds(w * RT, RT)])

        def advance(t, __):
            j = t // (K // 16)
            o = (t % (K // 16)) * 16
            idx_t[j, pl.ds(o, 16)] = idx_t[j, pl.ds(o, 16)] + 1
            return 0
        lax.fori_loop(0, ER4 * (K // 16), advance, 0)
        _zero_2d(gb0, K, F // 16)
        _zero_panel(gb0, z_sh, w * RT)
        plsc.subcore_barrier()
        return 0

    lax.fori_loop(0, P // NC, panel_body, 0)


def _sc_spmm_panels(u12, src2, dst2, ew2):
    kfn = pl.kernel(
        _spmm_body,
        out_type=[jax.ShapeDtypeStruct((P, NPAD, F), jnp.float32),
                  jax.ShapeDtypeStruct((NPAD,), jnp.float32)],
        mesh=plsc.VectorSubcoreMesh(**_MESH),
        compiler_params=_SC_PARAMS,
        scratch_types=[
            pltpu.VMEM((NPAD,), jnp.float32),
            pltpu.VMEM((ER4, K), jnp.int32),
            pltpu.VMEM((ER4, K), jnp.int32),
            pltpu.VMEM((ER4, K), jnp.float32),
            pltpu.VMEM((RT,), jnp.float32),
            pltpu.VMEM((K, F), jnp.float32),
            pltpu.VMEM((K, F), jnp.float32),
            pltpu.VMEM((K, F), jnp.float32),
            pltpu.VMEM((K, F), jnp.float32),
            pltpu.SemaphoreType.DMA,
            pltpu.SemaphoreType.DMA,
            pltpu.SemaphoreType.DMA,
            pltpu.SemaphoreType.DMA,
            pltpu.SemaphoreType.DMA,
            pltpu.SemaphoreType.DMA,
            pltpu.SemaphoreType.DMA,
            pltpu.SemaphoreType.DMA,
            pltpu.VMEM_SHARED((NPAD, F), jnp.float32),
            pltpu.VMEM_SHARED((NPAD,), jnp.float32),
        ],
    )
    return kfn(u12, src2, dst2, ew2)


def _spmm1_body(s_h, src_h, dst_h, ew_h, dis_h, out_h,
                dis_t, src_t, dst_t, norm_t,
                gb0, gb1, gb2, gb3, gs0, gs1, gs2, gs3,
                ss0, ss1, ss2, ss3, t_sh):
    c = lax.axis_index("c")
    w = lax.axis_index("s")
    row0 = (c * NS + w) * ER1
    pltpu.sync_copy(dis_h, dis_t)
    pltpu.sync_copy(src_h.at[pl.ds(row0, ER1)], src_t)
    pltpu.sync_copy(dst_h.at[pl.ds(row0, ER1)], dst_t)
    pltpu.sync_copy(ew_h.at[pl.ds(row0, ER1)], norm_t)
    _norm_inplace(dis_t, src_t, dst_t, norm_t, ER1)

    _zero_2d(gb0, K, F // 16)
    _zero_panel(gb0, t_sh, w * RT)
    plsc.subcore_barrier()
    _spmm_pipeline(s_h, src_t, dst_t, norm_t,
                   (gb0, gb1, gb2, gb3), (gs0, gs1, gs2, gs3),
                   (ss0, ss1, ss2, ss3), t_sh, ER1)
    plsc.subcore_barrier()
    pltpu.sync_copy(t_sh.at[pl.ds(w * RT, RT)],
                    out_h.at[c, pl.ds(w * RT, RT)])


def _sc_spmm_single(s, src2, dst2, ew2, dis):
    kfn = pl.kernel(
        _spmm1_body,
        out_type=jax.ShapeDtypeStruct((NC, NPAD, F), jnp.float32),
        mesh=plsc.VectorSubcoreMesh(**_MESH),
        compiler_params=_SC_PARAMS,
        scratch_types=[
            pltpu.VMEM((NPAD,), jnp.float32),
            pltpu.VMEM((ER1, K), jnp.int32),
            pltpu.VMEM((ER1, K), jnp.int32),
            pltpu.VMEM((ER1, K), jnp.float32),
            pltpu.VMEM((K, F), jnp.float32),
            pltpu.VMEM((K, F), jnp.float32),
            pltpu.VMEM((K, F), jnp.float32),
            pltpu.VMEM((K, F), jnp.float32),
            pltpu.SemaphoreType.DMA,
            pltpu.SemaphoreType.DMA,
            pltpu.SemaphoreType.DMA,
            pltpu.SemaphoreType.DMA,
            pltpu.SemaphoreType.DMA,
            pltpu.SemaphoreType.DMA,
            pltpu.SemaphoreType.DMA,
            pltpu.SemaphoreType.DMA,
            pltpu.VMEM_SHARED((NPAD, F), jnp.float32),
        ],
    )
    return kfn(s, src2, dst2, ew2, dis)


NB = 1000


def _k1_body(xt_ref, w_ref, u_ref):
    wm = w_ref[...]
    for p in range(P):
        y = lax.dot_general(xt_ref[p], wm, (((1,), (1,)), ((), ())),
                            preferred_element_type=jnp.float32)
        u_ref[:, p, :] = y


def _tc_proj(xt, w):
    return pl.pallas_call(
        _k1_body,
        grid=(N // NB,),
        in_specs=[
            pl.BlockSpec((P, NB, D), lambda i: (0, i, 0)),
            pl.BlockSpec((F, D), lambda i: (0, 0)),
        ],
        out_specs=pl.BlockSpec((NB, P, F), lambda i: (i, 0, 0)),
        out_shape=jax.ShapeDtypeStruct((N, P, F), jnp.float32),
    )(xt, w)


def _k5_body(v_ref, z_ref, b1_ref, s_ref):
    b1 = b1_ref[...]
    acc = jnp.zeros((NB, F), jnp.float32)
    for p in range(P):
        acc = acc + jax.nn.relu(v_ref[:, p, :] + z_ref[p] + b1)
    s_ref[...] = acc


def _tc_sum(v3, z12, b1):
    return pl.pallas_call(
        _k5_body,
        grid=(N // NB,),
        in_specs=[
            pl.BlockSpec((NB, P, F), lambda i: (i, 0, 0)),
            pl.BlockSpec((P, NB, F), lambda i: (0, i, 0)),
            pl.BlockSpec((1, F), lambda i: (0, 0)),
        ],
        out_specs=pl.BlockSpec((NB, F), lambda i: (i, 0)),
        out_shape=jax.ShapeDtypeStruct((N, F), jnp.float32),
    )(v3, z12, b1)


def _k7_body(s_ref, t_ref, w20_ref, w21_ref, b2_ref, l1w_ref, l1b_ref,
             l2w_ref, l2b_ref, h_ref, y_ref):
    s = s_ref[...]
    t = t_ref[0] + t_ref[1]
    hh = (lax.dot_general(s, w20_ref[...], (((1,), (1,)), ((), ())),
                          preferred_element_type=jnp.float32)
          + lax.dot_general(t, w21_ref[...], (((1,), (1,)), ((), ())),
                            preferred_element_type=jnp.float32)
          + P * b2_ref[...])
    a1 = jax.nn.relu(
        lax.dot_general(hh, l1w_ref[...], (((1,), (1,)), ((), ())),
                        preferred_element_type=jnp.float32) + l1b_ref[...])
    y = lax.dot_general(a1, l2w_ref[...], (((1,), (1,)), ((), ())),
                        preferred_element_type=jnp.float32) + l2b_ref[...]
    h_ref[...] = hh
    y_ref[...] = y


def _tc_head(s, t2, w20, w21, b2, l1w, l1b, l2w, l2b):
    def full(shape):
        return pl.BlockSpec(shape, lambda i, _s=shape: tuple(0 for _ in _s))
    return pl.pallas_call(
        _k7_body,
        grid=(N // NB,),
        in_specs=[
            pl.BlockSpec((NB, F), lambda i: (i, 0)),
            pl.BlockSpec((NC, NB, F), lambda i: (0, i, 0)),
            full((D, F)), full((D, F)), full((1, D)),
            full((128, D)), full((1, 128)),
            full((P, 128)), full((1, P)),
        ],
        out_specs=[
            pl.BlockSpec((NB, D), lambda i: (i, 0)),
            pl.BlockSpec((NB, P), lambda i: (i, 0)),
        ],
        out_shape=[
            jax.ShapeDtypeStruct((N, D), jnp.float32),
            jax.ShapeDtypeStruct((N, P), jnp.float32),
        ],
    )(s, t2, w20, w21, b2, l1w, l1b, l2w, l2b)


@jax.jit
def kernel(x, edge_index, edge_attr, W10, W11, b1, W20, W21, b2, L1w, L1b, L2w, L2b):
    src = edge_index[0].astype(jnp.int32)
    dst = edge_index[1].astype(jnp.int32)
    pad = EP - E
    pidx = (jnp.arange(pad, dtype=jnp.int32) % N)
    src2 = jnp.concatenate([src, pidx]).reshape(ECH, K)
    dst2 = jnp.concatenate([dst, pidx]).reshape(ECH, K)
    ew2 = jnp.concatenate([edge_attr, jnp.zeros((pad,), jnp.float32)]).reshape(ECH, K)

    xt = jnp.transpose(x, (2, 0, 1))
    u3 = _tc_proj(xt, W11)
    u12 = u3.reshape(P * N, F)

    z12, dis = _sc_spmm_panels(u12, src2, dst2, ew2)
    v3 = _tc_proj(xt, W10)
    s = _tc_sum(v3, z12, b1.reshape(1, F))
    t2 = _sc_spmm_single(s, src2, dst2, ew2, dis)
    h, y = _tc_head(s, t2, W20, W21, b2.reshape(1, D),
                    L1w, L1b.reshape(1, 128), L2w, L2b.reshape(1, P))
    return (y, h)
